# baseline (speedup 1.0000x reference)
import functools

import jax
import jax.numpy as jnp
from jax import lax
from jax.experimental import pallas as pl
from jax.experimental.pallas import tpu as pltpu
from jax.experimental.pallas import tpu_sc as plsc

N_VOCAB = 1_000_000
D_MODEL = 64
B_TOTAL = 4096 * 200
NUM_WORKERS = 32
B_PER_W = B_TOTAL // NUM_WORKERS
CHUNK = 800
N_CHUNKS = B_PER_W // CHUNK


def _body(table_hbm, idx_hbm, out_hbm, idx_v, rows_v, sem):
    wid = lax.axis_index("s") * 2 + lax.axis_index("c")
    base = wid * B_PER_W

    def step(g, carry):
        off = base + g * CHUNK
        pltpu.sync_copy(idx_hbm.at[pl.ds(off, CHUNK)], idx_v)
        pltpu.async_copy(table_hbm.at[idx_v], rows_v, sem).wait()
        pltpu.sync_copy(rows_v, out_hbm.at[pl.ds(off, CHUNK)])
        return carry

    lax.fori_loop(0, N_CHUNKS, step, 0)


@jax.jit
def _embed(weight, flat_idx):
    mesh = plsc.VectorSubcoreMesh(core_axis_name="c", subcore_axis_name="s")
    f = pl.kernel(
        _body,
        out_type=jax.ShapeDtypeStruct((B_TOTAL, D_MODEL), jnp.float32),
        mesh=mesh,
        scratch_types=[
            pltpu.VMEM((CHUNK,), jnp.int32),
            pltpu.VMEM((CHUNK, D_MODEL), jnp.float32),
            pltpu.SemaphoreType.DMA,
        ],
        compiler_params=pltpu.CompilerParams(use_tc_tiling_on_sc=False),
    )
    return f(weight, flat_idx)


def kernel(x, weight):
    flat = x.reshape(-1).astype(jnp.int32)
    out = _embed(weight, flat)
    return out.reshape(x.shape + (weight.shape[1],))


# speedup vs baseline: 1.0247x; 1.0247x over previous
import jax
import jax.numpy as jnp
from jax import lax
from jax.experimental import pallas as pl
from jax.experimental.pallas import tpu as pltpu
from jax.experimental.pallas import tpu_sc as plsc

N_VOCAB = 1_000_000
D_MODEL = 64
B_TOTAL = 4096 * 200
NUM_WORKERS = 32
B_PER_W = B_TOTAL // NUM_WORKERS
NSLOT = 4
CHUNK = 400
N_CHUNKS = B_PER_W // CHUNK
N_GROUPS = N_CHUNKS // NSLOT


def _body(table_hbm, idx_hbm, out_hbm, *refs):
    idx_v = refs[0:NSLOT]
    rows_v = refs[NSLOT:2 * NSLOT]
    isem = refs[2 * NSLOT:3 * NSLOT]
    gsem = refs[3 * NSLOT:4 * NSLOT]
    osem = refs[4 * NSLOT:5 * NSLOT]

    wid = lax.axis_index("s") * 2 + lax.axis_index("c")
    base = wid * B_PER_W

    def start_idx(g, b):
        pltpu.async_copy(idx_hbm.at[pl.ds(base + g * CHUNK, CHUNK)],
                         idx_v[b], isem[b])

    def start_gather(b):
        pltpu.async_copy(table_hbm.at[idx_v[b]], rows_v[b], gsem[b])

    def start_store(g, b):
        pltpu.async_copy(rows_v[b],
                         out_hbm.at[pl.ds(base + g * CHUNK, CHUNK)], osem[b])

    def wait_idx(b):
        pltpu.make_async_copy(idx_hbm.at[pl.ds(0, CHUNK)], idx_v[b],
                              isem[b]).wait()

    def wait_rows(sem, b):
        pltpu.make_async_copy(table_hbm.at[pl.ds(0, CHUNK)], rows_v[b],
                              sem[b]).wait()

    for b in range(NSLOT):
        start_idx(b, b)
    for b in range(NSLOT):
        wait_idx(b)
        start_gather(b)

    def group(p, carry):
        g0 = p * NSLOT
        for b in range(NSLOT):
            wait_rows(gsem, b)
            start_store(g0 + b, b)
            start_idx(g0 + b + NSLOT, b)
        for b in range(NSLOT):
            wait_idx(b)
            wait_rows(osem, b)
            start_gather(b)
        return carry

    lax.fori_loop(0, N_GROUPS - 1, group, 0)

    g0 = (N_GROUPS - 1) * NSLOT
    for b in range(NSLOT):
        wait_rows(gsem, b)
        start_store(g0 + b, b)
    for b in range(NSLOT):
        wait_rows(osem, b)


@jax.jit
def _embed(weight, flat_idx):
    mesh = plsc.VectorSubcoreMesh(core_axis_name="c", subcore_axis_name="s")
    scratch = (
        [pltpu.VMEM((CHUNK,), jnp.int32) for _ in range(NSLOT)]
        + [pltpu.VMEM((CHUNK, D_MODEL), jnp.float32) for _ in range(NSLOT)]
        + [pltpu.SemaphoreType.DMA for _ in range(3 * NSLOT)]
    )
    f = pl.kernel(
        _body,
        out_type=jax.ShapeDtypeStruct((B_TOTAL, D_MODEL), jnp.float32),
        mesh=mesh,
        scratch_types=scratch,
        compiler_params=pltpu.CompilerParams(use_tc_tiling_on_sc=False),
    )
    return f(weight, flat_idx)


def kernel(x, weight):
    flat = x.reshape(-1).astype(jnp.int32)
    out = _embed(weight, flat)
    return out.reshape(x.shape + (weight.shape[1],))


# speedup vs baseline: 1.3648x; 1.3319x over previous
import jax
import jax.numpy as jnp
from jax import lax
from jax.experimental import pallas as pl
from jax.experimental.pallas import tpu as pltpu
from jax.experimental.pallas import tpu_sc as plsc

N_VOCAB = 1_000_000
D_MODEL = 64
ROW = 128
N_BATCH = 4096
SEQ = 200
NUM_WORKERS = 32
ROWS_PER_W = N_BATCH // NUM_WORKERS
NSLOT = 4
N_GROUPS = ROWS_PER_W // NSLOT


def _body(table_hbm, idx_hbm, out_hbm, *refs):
    idx_v = refs[0:NSLOT]
    rows_v = refs[NSLOT:2 * NSLOT]
    isem = refs[2 * NSLOT:3 * NSLOT]
    gsem = refs[3 * NSLOT:4 * NSLOT]
    osem = refs[4 * NSLOT:5 * NSLOT]

    wid = lax.axis_index("s") * 2 + lax.axis_index("c")
    base = wid * ROWS_PER_W

    def start_idx(g, b):
        pltpu.async_copy(idx_hbm.at[pl.ds((base + g) * SEQ, SEQ)],
                         idx_v[b], isem[b])

    def start_gather(b):
        pltpu.async_copy(table_hbm.at[idx_v[b]], rows_v[b], gsem[b])

    def start_store(g, b):
        pltpu.async_copy(
            rows_v[b],
            out_hbm.at[base + g, pl.ds(0, SEQ), pl.ds(0, D_MODEL)],
            osem[b])

    def wait_idx(b):
        pltpu.make_async_copy(idx_hbm.at[pl.ds(0, SEQ)], idx_v[b],
                              isem[b]).wait()

    def wait_rows(sem, b):
        pltpu.make_async_copy(table_hbm.at[pl.ds(0, SEQ)], rows_v[b],
                              sem[b]).wait()

    for b in range(NSLOT):
        start_idx(b, b)
    for b in range(NSLOT):
        wait_idx(b)
        start_gather(b)

    def group(p, carry):
        g0 = p * NSLOT
        for b in range(NSLOT):
            wait_rows(gsem, b)
            start_store(g0 + b, b)
            start_idx(g0 + b + NSLOT, b)
        for b in range(NSLOT):
            wait_idx(b)
            wait_rows(osem, b)
            start_gather(b)
        return carry

    lax.fori_loop(0, N_GROUPS - 1, group, 0)

    g0 = (N_GROUPS - 1) * NSLOT
    for b in range(NSLOT):
        wait_rows(gsem, b)
        start_store(g0 + b, b)
    for b in range(NSLOT):
        wait_rows(osem, b)


@jax.jit
def _embed(weight_padded, flat_idx):
    mesh = plsc.VectorSubcoreMesh(core_axis_name="c", subcore_axis_name="s")
    scratch = (
        [pltpu.VMEM((SEQ,), jnp.int32) for _ in range(NSLOT)]
        + [pltpu.VMEM((SEQ, D_MODEL), jnp.float32) for _ in range(NSLOT)]
        + [pltpu.SemaphoreType.DMA for _ in range(3 * NSLOT)]
    )
    f = pl.kernel(
        _body,
        out_type=jax.ShapeDtypeStruct((N_BATCH, SEQ, ROW), jnp.float32),
        mesh=mesh,
        scratch_types=scratch,
        compiler_params=pltpu.CompilerParams(use_tc_tiling_on_sc=False),
    )
    return f(weight_padded, flat_idx)


def kernel(x, weight):
    flat = x.reshape(-1).astype(jnp.int32)
    w1d = lax.optimization_barrier(weight.reshape(-1))
    w2d = w1d.reshape(N_VOCAB, D_MODEL)
    out = _embed(w2d, flat)
    return lax.slice(out, (0, 0, 0), (N_BATCH, SEQ, D_MODEL))
